# Initial kernel scaffold; baseline (speedup 1.0000x reference)
import jax
import jax.numpy as jnp
from jax.experimental import pallas as pl


def kernel(inp, gate, weight1, weight2):
    raise NotImplementedError("write your pallas kernel here")


# speedup vs baseline: 48.0322x; 48.0322x over previous
import functools

import jax
import jax.numpy as jnp
from jax.experimental import pallas as pl
from jax.experimental.pallas import tpu as pltpu

NUM_E = 16
D_IN = 1024
D_HID = 4096
D_OUT = 1024
T = 1024

C = 64
BH = 512
NH = D_HID // BH


def _moe_tc_body(offs_ref, x_ref, w1_ref, w2_ref, out_ref):
    e = pl.program_id(0)
    h = pl.program_id(1)

    @pl.when((e == 0) & (h == 0))
    def _init():
        out_ref[...] = jnp.zeros_like(out_ref)

    start = offs_ref[e]
    end = offs_ref[e + 1]
    j0 = start // C
    j1 = (end + C - 1) // C

    w1b = w1_ref[0]
    w2b = w2_ref[0]

    def body(j, carry):
        base = j * C
        xs = x_ref[pl.ds(base, C), :]
        hid = jax.lax.dot_general(
            xs, w1b, (((1,), (1,)), ((), ())),
            preferred_element_type=jnp.float32)
        contrib = jax.lax.dot_general(
            hid, w2b, (((1,), (1,)), ((), ())),
            preferred_element_type=jnp.float32)
        rows = base + jax.lax.broadcasted_iota(jnp.int32, (C, 1), 0)
        valid = (rows >= start) & (rows < end)
        contrib = jnp.where(valid, contrib, 0.0)
        out_ref[pl.ds(base, C), :] += contrib
        return carry

    jax.lax.fori_loop(j0, j1, body, 0)


@functools.partial(jax.jit, static_argnames=("interpret",))
def _grouped_gemm(offs, x_sorted, weight1, weight2, interpret=False):
    return pl.pallas_call(
        _moe_tc_body,
        grid=(NUM_E, NH),
        in_specs=[
            pl.BlockSpec(memory_space=pltpu.SMEM),
            pl.BlockSpec((T, D_IN), lambda e, h: (0, 0)),
            pl.BlockSpec((1, BH, D_IN), lambda e, h: (e, h, 0)),
            pl.BlockSpec((1, D_OUT, BH), lambda e, h: (e, 0, h)),
        ],
        out_specs=pl.BlockSpec((T, D_OUT), lambda e, h: (0, 0)),
        out_shape=jax.ShapeDtypeStruct((T, D_OUT), jnp.float32),
        interpret=interpret,
    )(offs, x_sorted, weight1, weight2)


def kernel(inp, gate, weight1, weight2, interpret=False):
    gate = gate.astype(jnp.int32)
    sidx = jnp.argsort(gate, stable=True)
    pos = jnp.argsort(sidx, stable=True)
    counts = jnp.zeros((NUM_E,), jnp.int32).at[gate].add(1)
    offs = jnp.concatenate(
        [jnp.zeros((1,), jnp.int32), jnp.cumsum(counts, dtype=jnp.int32)])
    x_sorted = inp[sidx]
    out_sorted = _grouped_gemm(offs, x_sorted, weight1, weight2,
                               interpret=interpret)
    return out_sorted[pos]
